# baseline (speedup 1.0000x reference)
import functools
import jax
import jax.numpy as jnp
from jax import lax
from jax.experimental import pallas as pl
from jax.experimental.pallas import tpu as pltpu
from jax.experimental.pallas import tpu_sc as plsc

N_SRC = 40000
N_DST = 10000
E = 320000
D = 128
H = 128

NC = 2
NS = 16
NW = NC * NS
EPW = E // NW
CH = 80
NCHUNK = EPW // CH
NPAIR = (NCHUNK + 1) // 2
NZCH = N_DST // CH
NZROUND = -(-NZCH // NS)

_LANES = 16


def _silu(x):
    return x * jax.nn.sigmoid(x)


def _ln(y, s, b):
    mu = jnp.mean(y, axis=-1, keepdims=True)
    var = jnp.mean((y - mu) ** 2, axis=-1, keepdims=True)
    return (y - mu) / jnp.sqrt(var + 1e-5) * s + b


def _proj_body(x1_ref, x2_ref, ws_ref, wd_ref, p_ref, q_ref):
    dn = (((1,), (1,)), ((), ()))
    p_ref[...] = lax.dot_general(x1_ref[...], ws_ref[...], dn,
                                 preferred_element_type=jnp.float32)
    q_ref[...] = lax.dot_general(x2_ref[...], wd_ref[...], dn,
                                 preferred_element_type=jnp.float32)


def _proj(x1, x2, ws, wd):
    return pl.pallas_call(
        _proj_body,
        out_shape=[jax.ShapeDtypeStruct((N_DST, H), jnp.float32),
                   jax.ShapeDtypeStruct((N_DST, H), jnp.float32)],
    )(x1, x2, ws, wd)


NBUF = 5
NGRP = NCHUNK // NBUF


def _gather_body(sidx_hbm, didx_hbm, p_hbm, q_hbm, g_hbm,
                 sall, dall, *bufs):
    pbs = bufs[0:NBUF]
    qbs = bufs[NBUF:2 * NBUF]
    sps = bufs[2 * NBUF:3 * NBUF]
    sqs = bufs[3 * NBUF:4 * NBUF]
    sws = bufs[4 * NBUF:5 * NBUF]
    wid = lax.axis_index("s") * NC + lax.axis_index("c")
    base = pl.multiple_of(wid * EPW, 8)
    pltpu.sync_copy(sidx_hbm.at[pl.ds(base, EPW)], sall)
    pltpu.sync_copy(didx_hbm.at[pl.ds(base, EPW)], dall)

    def start(c, b):
        sl = pl.ds(pl.multiple_of(c * CH, 8), CH)
        pltpu.async_copy(q_hbm.at[dall.at[sl]], qbs[b], sqs[b])

    def waitg(b):
        pltpu.make_async_copy(p_hbm.at[pl.ds(0, CH)], pbs[b], sps[b]).wait()
        pltpu.make_async_copy(q_hbm.at[pl.ds(0, CH)], qbs[b], sqs[b]).wait()

    def waitw(b):
        pltpu.make_async_copy(qbs[b], g_hbm.at[pl.ds(0, CH)], sws[b]).wait()

    for b in range(NBUF - 1):
        start(b, b)

    def it(k, carry):
        c0 = k * NBUF
        for b in range(NBUF):
            c = c0 + b
            sl = pl.ds(pl.multiple_of(c * CH, 8), CH)
            pltpu.make_async_copy(
                q_hbm.at[pl.ds(0, CH)], qbs[b], sqs[b]).wait()
            pltpu.async_copy(p_hbm.at[sall.at[sl]], qbs[b], sps[b],
                             add=True)
            pltpu.make_async_copy(
                q_hbm.at[pl.ds(0, CH)], qbs[b], sps[b]).wait()
            pltpu.async_copy(
                qbs[b],
                g_hbm.at[pl.ds(pl.multiple_of(base + c * CH, 8), CH)],
                sws[b])
            nb = (b + NBUF - 1) % NBUF

            @pl.when(c + NBUF - 1 < NCHUNK)
            def _():
                @pl.when(c >= 1)
                def _():
                    waitw(nb)

                start(c + NBUF - 1, nb)

        return carry

    lax.fori_loop(0, NGRP, it, 0)
    for b in range(NBUF):
        waitw(b)


def _gather_add(src_idx, dst_idx, p, q):
    mesh = plsc.VectorSubcoreMesh(core_axis_name="c", subcore_axis_name="s")
    fn = functools.partial(
        pl.kernel,
        mesh=mesh,
        out_type=jax.ShapeDtypeStruct((E, D), jnp.float32),
        scratch_types=(
            [pltpu.VMEM((EPW,), jnp.int32)] * 2
            + [pltpu.VMEM((CH, D), jnp.float32)] * (2 * NBUF)
            + [pltpu.SemaphoreType.DMA] * (3 * NBUF)
        ),
    )(_gather_body)
    return fn(src_idx, dst_idx, p, q)


BE = 8000


def _edge_body(e_ref, g_ref, we_ref, be1_ref, we2_ref, be2_ref,
               s_ref, b_ref, o_ref):
    dn = (((1,), (1,)), ((), ()))
    h = lax.dot_general(e_ref[...], we_ref[...], dn,
                        preferred_element_type=jnp.float32)
    h = h + g_ref[...] + be1_ref[...]
    h = _silu(h)
    y = lax.dot_general(h, we2_ref[...], dn,
                        preferred_element_type=jnp.float32) + be2_ref[...]
    o_ref[...] = _ln(y, s_ref[...], b_ref[...])


def _edge_mlp(e, g, we, be1, we2, be2, eln_s, eln_b):
    grid = (E // BE,)
    row_spec = pl.BlockSpec((BE, D), lambda i: (i, 0))
    w_spec = pl.BlockSpec((H, D), lambda i: (0, 0))
    v_spec = pl.BlockSpec((1, D), lambda i: (0, 0))
    return pl.pallas_call(
        _edge_body,
        grid=grid,
        in_specs=[row_spec, row_spec, w_spec, v_spec,
                  pl.BlockSpec((D, H), lambda i: (0, 0)), v_spec,
                  v_spec, v_spec],
        out_specs=row_spec,
        out_shape=jax.ShapeDtypeStruct((E, D), jnp.float32),
    )(e, g, we, be1, we2, be2, eln_s, eln_b)


NBUF_S = 4
NGRP_S = -(-NCHUNK // NBUF_S)


def _scatter_body(didx_hbm, ef_hbm, out_hbm, *bufs):
    ibs = bufs[0:NBUF_S]
    rbs = bufs[NBUF_S:2 * NBUF_S]
    agg_sh = bufs[2 * NBUF_S]
    sis = bufs[2 * NBUF_S + 1:3 * NBUF_S + 1]
    srs = bufs[3 * NBUF_S + 1:4 * NBUF_S + 1]
    sas = bufs[4 * NBUF_S + 1:5 * NBUF_S + 1]
    cid = lax.axis_index("c")
    sid = lax.axis_index("s")
    wid = sid * NC + cid
    base = pl.multiple_of(wid * EPW, 8)
    rb0 = rbs[0]

    def zrow(r, carry):
        for j in range(D // _LANES):
            rb0[r, pl.ds(j * _LANES, _LANES)] = jnp.zeros((_LANES,),
                                                          jnp.float32)
        return carry

    lax.fori_loop(0, CH, zrow, 0)

    def zchunk(c, carry):
        zc = c * NS + sid

        @pl.when(zc < NZCH)
        def _():
            pltpu.sync_copy(
                rb0, agg_sh.at[pl.ds(pl.multiple_of(zc * CH, 8), CH)])

        return carry

    lax.fori_loop(0, NZROUND, zchunk, 0)
    plsc.subcore_barrier()

    def start(c, b):
        off = pl.multiple_of(base + c * CH, 8)
        pltpu.async_copy(didx_hbm.at[pl.ds(off, CH)], ibs[b], sis[b])
        pltpu.async_copy(ef_hbm.at[pl.ds(off, CH)], rbs[b], srs[b])

    for b in range(NBUF_S - 1):
        start(b, b)

    def it(k, carry):
        c0 = k * NBUF_S
        for b in range(NBUF_S):
            c = c0 + b

            @pl.when(c < NCHUNK)
            def _():
                pltpu.make_async_copy(
                    didx_hbm.at[pl.ds(0, CH)], ibs[b], sis[b]).wait()
                pltpu.make_async_copy(
                    ef_hbm.at[pl.ds(0, CH)], rbs[b], srs[b]).wait()
                pltpu.async_copy(rbs[b], agg_sh.at[ibs[b]], sas[b],
                                 add=True)

            nb = (b + NBUF_S - 1) % NBUF_S

            @pl.when(c + NBUF_S - 1 < NCHUNK)
            def _():
                @pl.when(c >= 1)
                def _():
                    pltpu.make_async_copy(
                        rbs[nb], agg_sh.at[pl.ds(0, CH)], sas[nb]).wait()

                start(c + NBUF_S - 1, nb)

        return carry

    lax.fori_loop(0, NGRP_S, it, 0)
    for b in range(NBUF_S):
        pltpu.make_async_copy(
            rbs[b], agg_sh.at[pl.ds(0, CH)], sas[b]).wait()
    plsc.subcore_barrier()

    def wchunk(c, carry):
        zc = c * NS + sid

        @pl.when(zc < NZCH)
        def _():
            off = pl.multiple_of(zc * CH, 8)
            pltpu.sync_copy(agg_sh.at[pl.ds(off, CH)], rb0)
            pltpu.sync_copy(rb0, out_hbm.at[cid, pl.ds(off, CH)])

        return carry

    lax.fori_loop(0, NZROUND, wchunk, 0)


def _scatter_agg(dst_idx, ef):
    mesh = plsc.VectorSubcoreMesh(core_axis_name="c", subcore_axis_name="s")
    fn = functools.partial(
        pl.kernel,
        mesh=mesh,
        out_type=jax.ShapeDtypeStruct((NC, N_DST, D), jnp.float32),
        scratch_types=(
            [pltpu.VMEM((CH,), jnp.int32)] * NBUF_S
            + [pltpu.VMEM((CH, D), jnp.float32)] * NBUF_S
            + [pltpu.VMEM_SHARED((N_DST, D), jnp.float32)]
            + [pltpu.SemaphoreType.DMA] * (3 * NBUF_S)
        ),
    )(_scatter_body)
    return fn(dst_idx, ef)


BD = 2000


def _dst_body(p0_ref, p1_ref, m_ref, wa_ref, wb_ref, bd1_ref,
              wd2_ref, bd2_ref, s_ref, b_ref, o_ref):
    dn = (((1,), (1,)), ((), ()))
    agg = p0_ref[...] + p1_ref[...]
    m = m_ref[...]
    hd = lax.dot_general(agg, wa_ref[...], dn,
                         preferred_element_type=jnp.float32)
    hd = hd + lax.dot_general(m, wb_ref[...], dn,
                              preferred_element_type=jnp.float32)
    hd = _silu(hd + bd1_ref[...])
    y = lax.dot_general(hd, wd2_ref[...], dn,
                        preferred_element_type=jnp.float32) + bd2_ref[...]
    o_ref[...] = m + _ln(y, s_ref[...], b_ref[...])


def _dst_mlp(p0, p1, m2m, wa, wb, bd1, wd2, bd2, dln_s, dln_b):
    grid = (N_DST // BD,)
    row_spec = pl.BlockSpec((BD, D), lambda i: (i, 0))
    w_spec = pl.BlockSpec((H, D), lambda i: (0, 0))
    v_spec = pl.BlockSpec((1, D), lambda i: (0, 0))
    return pl.pallas_call(
        _dst_body,
        grid=grid,
        in_specs=[row_spec, row_spec, row_spec, w_spec, w_spec, v_spec,
                  pl.BlockSpec((D, H), lambda i: (0, 0)), v_spec,
                  v_spec, v_spec],
        out_specs=row_spec,
        out_shape=jax.ShapeDtypeStruct((N_DST, D), jnp.float32),
    )(p0, p1, m2m, wa, wb, bd1, wd2, bd2, dln_s, dln_b)


BS = 4000


def _src_body(x_ref, w1_ref, b1_ref, w2_ref, b2_ref, s_ref, b_ref, o_ref):
    dn = (((1,), (1,)), ((), ()))
    x = x_ref[...]
    h = _silu(lax.dot_general(x, w1_ref[...], dn,
                              preferred_element_type=jnp.float32)
              + b1_ref[...])
    y = lax.dot_general(h, w2_ref[...], dn,
                        preferred_element_type=jnp.float32) + b2_ref[...]
    o_ref[...] = x + _ln(y, s_ref[...], b_ref[...])


def _src_mlp(x, w1, b1, w2, b2, sln_s, sln_b):
    grid = (N_SRC // BS,)
    row_spec = pl.BlockSpec((BS, D), lambda i: (i, 0))
    w_spec = pl.BlockSpec((H, D), lambda i: (0, 0))
    v_spec = pl.BlockSpec((1, D), lambda i: (0, 0))
    return pl.pallas_call(
        _src_body,
        grid=grid,
        in_specs=[row_spec, w_spec, v_spec,
                  pl.BlockSpec((D, H), lambda i: (0, 0)), v_spec,
                  v_spec, v_spec],
        out_specs=row_spec,
        out_shape=jax.ShapeDtypeStruct((N_SRC, D), jnp.float32),
    )(x, w1, b1, w2, b2, sln_s, sln_b)


def kernel(g2m_graph, grid_embedded, m2m_node_embedded, g2m_edge_embedded,
           We, Ws, Wd, be1, We2, be2, eln_s, eln_b,
           Ws1, bs1, Ws2, bs2, sln_s, sln_b,
           Wd1, bd1, Wd2, bd2, dln_s, dln_b):
    src_idx = g2m_graph[0]
    dst_idx = g2m_graph[1]
    r = lambda v: v.reshape(1, -1)

    p, q = _proj(grid_embedded[:N_DST], m2m_node_embedded, Ws, Wd)
    g = _gather_add(src_idx, dst_idx, p, q)
    ef = _edge_mlp(g2m_edge_embedded, g, We, r(be1), We2, r(be2),
                   r(eln_s), r(eln_b))
    partials = _scatter_agg(dst_idx, ef)
    m2m_out = _dst_mlp(partials[0], partials[1], m2m_node_embedded,
                       Wd1[:, :D], Wd1[:, D:], r(bd1), Wd2, r(bd2),
                       r(dln_s), r(dln_b))
    grid_out = _src_mlp(grid_embedded, Ws1, r(bs1), Ws2, r(bs2),
                        r(sln_s), r(sln_b))
    return (grid_out, m2m_out)


# speedup vs baseline: 1.0522x; 1.0522x over previous
import functools
import jax
import jax.numpy as jnp
from jax import lax
from jax.experimental import pallas as pl
from jax.experimental.pallas import tpu as pltpu
from jax.experimental.pallas import tpu_sc as plsc

N_SRC = 40000
N_DST = 10000
E = 320000
D = 128
H = 128

NC = 2
NS = 16
NW = NC * NS
EPW = E // NW
CH = 80
NCHUNK = EPW // CH
NPAIR = (NCHUNK + 1) // 2
NZCH = N_DST // CH
NZROUND = -(-NZCH // NS)

_LANES = 16


def _silu(x):
    return x * jax.nn.sigmoid(x)


def _ln(y, s, b):
    mu = jnp.mean(y, axis=-1, keepdims=True)
    var = jnp.mean((y - mu) ** 2, axis=-1, keepdims=True)
    return (y - mu) / jnp.sqrt(var + 1e-5) * s + b


def _proj_body(x1_ref, x2_ref, ws_ref, wd_ref, p_ref, q_ref):
    dn = (((1,), (1,)), ((), ()))
    p_ref[...] = lax.dot_general(x1_ref[...], ws_ref[...], dn,
                                 preferred_element_type=jnp.float32)
    q_ref[...] = lax.dot_general(x2_ref[...], wd_ref[...], dn,
                                 preferred_element_type=jnp.float32)


def _proj(x1, x2, ws, wd):
    return pl.pallas_call(
        _proj_body,
        out_shape=[jax.ShapeDtypeStruct((N_DST, H), jnp.float32),
                   jax.ShapeDtypeStruct((N_DST, H), jnp.float32)],
    )(x1, x2, ws, wd)


NBUF = 5
NGRP = NCHUNK // NBUF


def _gather_body(sidx_hbm, didx_hbm, p_hbm, q_hbm, g_hbm,
                 sall, dall, *bufs):
    pbs = bufs[0:NBUF]
    qbs = bufs[NBUF:2 * NBUF]
    sps = bufs[2 * NBUF:3 * NBUF]
    sqs = bufs[3 * NBUF:4 * NBUF]
    sws = bufs[4 * NBUF:5 * NBUF]
    wid = lax.axis_index("s") * NC + lax.axis_index("c")
    base = pl.multiple_of(wid * EPW, 8)
    pltpu.sync_copy(sidx_hbm.at[pl.ds(base, EPW)], sall)
    pltpu.sync_copy(didx_hbm.at[pl.ds(base, EPW)], dall)

    def start_q(c, b):
        sl = pl.ds(pl.multiple_of(c * CH, 8), CH)
        pltpu.async_copy(q_hbm.at[dall.at[sl]], qbs[b], sqs[b])

    def start_padd(c, b):
        sl = pl.ds(pl.multiple_of(c * CH, 8), CH)
        pltpu.async_copy(p_hbm.at[sall.at[sl]], qbs[b], sps[b], add=True)

    def start_store(c, b):
        pltpu.async_copy(
            qbs[b], g_hbm.at[pl.ds(pl.multiple_of(base + c * CH, 8), CH)],
            sws[b])

    def waitq(b):
        pltpu.make_async_copy(q_hbm.at[pl.ds(0, CH)], qbs[b], sqs[b]).wait()

    def waitp(b):
        pltpu.make_async_copy(q_hbm.at[pl.ds(0, CH)], qbs[b], sps[b]).wait()

    def waitw(b):
        pltpu.make_async_copy(qbs[b], g_hbm.at[pl.ds(0, CH)], sws[b]).wait()

    for b in range(3):
        start_q(b, b)

    def it(k, carry):
        c0 = k * NBUF
        for b in range(NBUF):
            c = c0 + b
            waitq(b)
            start_padd(c, b)
            pb2 = (b + NBUF - 1) % NBUF

            @pl.when(c >= 1)
            def _():
                waitp(pb2)
                start_store(c - 1, pb2)

            nb = (b + 3) % NBUF

            @pl.when(c + 3 < NCHUNK)
            def _():
                @pl.when(c >= 2)
                def _():
                    waitw(nb)

                start_q(c + 3, nb)

        return carry

    lax.fori_loop(0, NGRP, it, 0)
    waitp(NBUF - 1)
    start_store(NCHUNK - 1, NBUF - 1)
    for b in range(NBUF):
        waitw(b)


def _gather_add(src_idx, dst_idx, p, q):
    mesh = plsc.VectorSubcoreMesh(core_axis_name="c", subcore_axis_name="s")
    fn = functools.partial(
        pl.kernel,
        mesh=mesh,
        out_type=jax.ShapeDtypeStruct((E, D), jnp.float32),
        scratch_types=(
            [pltpu.VMEM((EPW,), jnp.int32)] * 2
            + [pltpu.VMEM((CH, D), jnp.float32)] * (2 * NBUF)
            + [pltpu.SemaphoreType.DMA] * (3 * NBUF)
        ),
    )(_gather_body)
    return fn(src_idx, dst_idx, p, q)


BE = 8000


def _edge_body(e_ref, g_ref, we_ref, be1_ref, we2_ref, be2_ref,
               s_ref, b_ref, o_ref):
    dn = (((1,), (1,)), ((), ()))
    h = lax.dot_general(e_ref[...], we_ref[...], dn,
                        preferred_element_type=jnp.float32)
    h = h + g_ref[...] + be1_ref[...]
    h = _silu(h)
    y = lax.dot_general(h, we2_ref[...], dn,
                        preferred_element_type=jnp.float32) + be2_ref[...]
    o_ref[...] = _ln(y, s_ref[...], b_ref[...])


def _edge_mlp(e, g, we, be1, we2, be2, eln_s, eln_b):
    grid = (E // BE,)
    row_spec = pl.BlockSpec((BE, D), lambda i: (i, 0))
    w_spec = pl.BlockSpec((H, D), lambda i: (0, 0))
    v_spec = pl.BlockSpec((1, D), lambda i: (0, 0))
    return pl.pallas_call(
        _edge_body,
        grid=grid,
        in_specs=[row_spec, row_spec, w_spec, v_spec,
                  pl.BlockSpec((D, H), lambda i: (0, 0)), v_spec,
                  v_spec, v_spec],
        out_specs=row_spec,
        out_shape=jax.ShapeDtypeStruct((E, D), jnp.float32),
    )(e, g, we, be1, we2, be2, eln_s, eln_b)


NBUF_S = 4
NGRP_S = -(-NCHUNK // NBUF_S)


def _scatter_body(didx_hbm, ef_hbm, out_hbm, *bufs):
    ibs = bufs[0:NBUF_S]
    rbs = bufs[NBUF_S:2 * NBUF_S]
    agg_sh = bufs[2 * NBUF_S]
    sis = bufs[2 * NBUF_S + 1:3 * NBUF_S + 1]
    srs = bufs[3 * NBUF_S + 1:4 * NBUF_S + 1]
    sas = bufs[4 * NBUF_S + 1:5 * NBUF_S + 1]
    cid = lax.axis_index("c")
    sid = lax.axis_index("s")
    wid = sid * NC + cid
    base = pl.multiple_of(wid * EPW, 8)
    rb0 = rbs[0]

    def zrow(r, carry):
        for j in range(D // _LANES):
            rb0[r, pl.ds(j * _LANES, _LANES)] = jnp.zeros((_LANES,),
                                                          jnp.float32)
        return carry

    lax.fori_loop(0, CH, zrow, 0)

    def zchunk(c, carry):
        zc = c * NS + sid

        @pl.when(zc < NZCH)
        def _():
            pltpu.sync_copy(
                rb0, agg_sh.at[pl.ds(pl.multiple_of(zc * CH, 8), CH)])

        return carry

    lax.fori_loop(0, NZROUND, zchunk, 0)
    plsc.subcore_barrier()

    def start(c, b):
        off = pl.multiple_of(base + c * CH, 8)
        pltpu.async_copy(didx_hbm.at[pl.ds(off, CH)], ibs[b], sis[b])
        pltpu.async_copy(ef_hbm.at[pl.ds(off, CH)], rbs[b], srs[b])

    for b in range(NBUF_S - 1):
        start(b, b)

    def it(k, carry):
        c0 = k * NBUF_S
        for b in range(NBUF_S):
            c = c0 + b

            @pl.when(c < NCHUNK)
            def _():
                pltpu.make_async_copy(
                    didx_hbm.at[pl.ds(0, CH)], ibs[b], sis[b]).wait()
                pltpu.make_async_copy(
                    ef_hbm.at[pl.ds(0, CH)], rbs[b], srs[b]).wait()
                pltpu.async_copy(rbs[b], agg_sh.at[ibs[b]], sas[b],
                                 add=True)

            nb = (b + NBUF_S - 1) % NBUF_S

            @pl.when(c + NBUF_S - 1 < NCHUNK)
            def _():
                @pl.when(c >= 1)
                def _():
                    pltpu.make_async_copy(
                        rbs[nb], agg_sh.at[pl.ds(0, CH)], sas[nb]).wait()

                start(c + NBUF_S - 1, nb)

        return carry

    lax.fori_loop(0, NGRP_S, it, 0)
    for b in range(NBUF_S):
        pltpu.make_async_copy(
            rbs[b], agg_sh.at[pl.ds(0, CH)], sas[b]).wait()
    plsc.subcore_barrier()

    def wchunk(c, carry):
        zc = c * NS + sid

        @pl.when(zc < NZCH)
        def _():
            off = pl.multiple_of(zc * CH, 8)
            pltpu.sync_copy(agg_sh.at[pl.ds(off, CH)], rb0)
            pltpu.sync_copy(rb0, out_hbm.at[cid, pl.ds(off, CH)])

        return carry

    lax.fori_loop(0, NZROUND, wchunk, 0)


def _scatter_agg(dst_idx, ef):
    mesh = plsc.VectorSubcoreMesh(core_axis_name="c", subcore_axis_name="s")
    fn = functools.partial(
        pl.kernel,
        mesh=mesh,
        out_type=jax.ShapeDtypeStruct((NC, N_DST, D), jnp.float32),
        scratch_types=(
            [pltpu.VMEM((CH,), jnp.int32)] * NBUF_S
            + [pltpu.VMEM((CH, D), jnp.float32)] * NBUF_S
            + [pltpu.VMEM_SHARED((N_DST, D), jnp.float32)]
            + [pltpu.SemaphoreType.DMA] * (3 * NBUF_S)
        ),
    )(_scatter_body)
    return fn(dst_idx, ef)


BD = 2000


def _dst_body(p0_ref, p1_ref, m_ref, wa_ref, wb_ref, bd1_ref,
              wd2_ref, bd2_ref, s_ref, b_ref, o_ref):
    dn = (((1,), (1,)), ((), ()))
    agg = p0_ref[...] + p1_ref[...]
    m = m_ref[...]
    hd = lax.dot_general(agg, wa_ref[...], dn,
                         preferred_element_type=jnp.float32)
    hd = hd + lax.dot_general(m, wb_ref[...], dn,
                              preferred_element_type=jnp.float32)
    hd = _silu(hd + bd1_ref[...])
    y = lax.dot_general(hd, wd2_ref[...], dn,
                        preferred_element_type=jnp.float32) + bd2_ref[...]
    o_ref[...] = m + _ln(y, s_ref[...], b_ref[...])


def _dst_mlp(p0, p1, m2m, wa, wb, bd1, wd2, bd2, dln_s, dln_b):
    grid = (N_DST // BD,)
    row_spec = pl.BlockSpec((BD, D), lambda i: (i, 0))
    w_spec = pl.BlockSpec((H, D), lambda i: (0, 0))
    v_spec = pl.BlockSpec((1, D), lambda i: (0, 0))
    return pl.pallas_call(
        _dst_body,
        grid=grid,
        in_specs=[row_spec, row_spec, row_spec, w_spec, w_spec, v_spec,
                  pl.BlockSpec((D, H), lambda i: (0, 0)), v_spec,
                  v_spec, v_spec],
        out_specs=row_spec,
        out_shape=jax.ShapeDtypeStruct((N_DST, D), jnp.float32),
    )(p0, p1, m2m, wa, wb, bd1, wd2, bd2, dln_s, dln_b)


BS = 4000


def _src_body(x_ref, w1_ref, b1_ref, w2_ref, b2_ref, s_ref, b_ref, o_ref):
    dn = (((1,), (1,)), ((), ()))
    x = x_ref[...]
    h = _silu(lax.dot_general(x, w1_ref[...], dn,
                              preferred_element_type=jnp.float32)
              + b1_ref[...])
    y = lax.dot_general(h, w2_ref[...], dn,
                        preferred_element_type=jnp.float32) + b2_ref[...]
    o_ref[...] = x + _ln(y, s_ref[...], b_ref[...])


def _src_mlp(x, w1, b1, w2, b2, sln_s, sln_b):
    grid = (N_SRC // BS,)
    row_spec = pl.BlockSpec((BS, D), lambda i: (i, 0))
    w_spec = pl.BlockSpec((H, D), lambda i: (0, 0))
    v_spec = pl.BlockSpec((1, D), lambda i: (0, 0))
    return pl.pallas_call(
        _src_body,
        grid=grid,
        in_specs=[row_spec, w_spec, v_spec,
                  pl.BlockSpec((D, H), lambda i: (0, 0)), v_spec,
                  v_spec, v_spec],
        out_specs=row_spec,
        out_shape=jax.ShapeDtypeStruct((N_SRC, D), jnp.float32),
    )(x, w1, b1, w2, b2, sln_s, sln_b)


def kernel(g2m_graph, grid_embedded, m2m_node_embedded, g2m_edge_embedded,
           We, Ws, Wd, be1, We2, be2, eln_s, eln_b,
           Ws1, bs1, Ws2, bs2, sln_s, sln_b,
           Wd1, bd1, Wd2, bd2, dln_s, dln_b):
    src_idx = g2m_graph[0]
    dst_idx = g2m_graph[1]
    r = lambda v: v.reshape(1, -1)

    p, q = _proj(grid_embedded[:N_DST], m2m_node_embedded, Ws, Wd)
    g = _gather_add(src_idx, dst_idx, p, q)
    ef = _edge_mlp(g2m_edge_embedded, g, We, r(be1), We2, r(be2),
                   r(eln_s), r(eln_b))
    partials = _scatter_agg(dst_idx, ef)
    m2m_out = _dst_mlp(partials[0], partials[1], m2m_node_embedded,
                       Wd1[:, :D], Wd1[:, D:], r(bd1), Wd2, r(bd2),
                       r(dln_s), r(dln_b))
    grid_out = _src_mlp(grid_embedded, Ws1, r(bs1), Ws2, r(bs2),
                        r(sln_s), r(sln_b))
    return (grid_out, m2m_out)
